# initial kernel scaffold (unmeasured)
import jax
import jax.numpy as jnp
from jax import lax
from jax.experimental import pallas as pl
from jax.experimental.pallas import tpu as pltpu


def kernel(
    x,
):
    def body(*refs):
        pass

    out_shape = jax.ShapeDtypeStruct(..., jnp.float32)
    return pl.pallas_call(body, out_shape=out_shape)(...)


# baseline (device time: 754210 ns/iter reference)
import jax
import jax.numpy as jnp
from jax import lax
from jax.experimental import pallas as pl
from jax.experimental.pallas import tpu as pltpu

N_DEV = 8


def kernel(x):
    m_per, n = x.shape
    half = m_per // 2

    def body(x_ref, out_ref, stage_ref, copy_sem,
             send_cw, recv_cw, send_ccw, recv_ccw):
        my = lax.axis_index("i")
        left = lax.rem(my + N_DEV - 1, N_DEV)
        right = lax.rem(my + 1, N_DEV)

        barrier = pltpu.get_barrier_semaphore()
        for nbr in (left, right):
            pl.semaphore_signal(barrier, inc=1, device_id=(nbr,),
                                device_id_type=pl.DeviceIdType.MESH)
        pl.semaphore_wait(barrier, 2)

        stage_ref[...] = x_ref[...].astype(out_ref.dtype)
        local = pltpu.make_async_copy(
            stage_ref, out_ref.at[pl.ds(my * m_per, m_per), :], copy_sem)
        local.start()
        local.wait()

        for h in range(N_DEV - 1):
            o_cw = lax.rem(my - h + N_DEV, N_DEV)
            o_ccw = lax.rem(my + h, N_DEV)
            rdma_cw = pltpu.make_async_remote_copy(
                src_ref=out_ref.at[pl.ds(o_cw * m_per, half), :],
                dst_ref=out_ref.at[pl.ds(o_cw * m_per, half), :],
                send_sem=send_cw.at[h], recv_sem=recv_cw.at[h],
                device_id=(right,), device_id_type=pl.DeviceIdType.MESH)
            rdma_ccw = pltpu.make_async_remote_copy(
                src_ref=out_ref.at[pl.ds(o_ccw * m_per + half, half), :],
                dst_ref=out_ref.at[pl.ds(o_ccw * m_per + half, half), :],
                send_sem=send_ccw.at[h], recv_sem=recv_ccw.at[h],
                device_id=(left,), device_id_type=pl.DeviceIdType.MESH)
            rdma_cw.start()
            rdma_ccw.start()
            rdma_cw.wait()
            rdma_ccw.wait()

    return pl.pallas_call(
        body,
        out_shape=jax.ShapeDtypeStruct((N_DEV * m_per, n), jnp.bfloat16),
        in_specs=[pl.BlockSpec(memory_space=pltpu.VMEM)],
        out_specs=pl.BlockSpec(memory_space=pl.ANY),
        scratch_shapes=[
            pltpu.VMEM((m_per, n), jnp.bfloat16),
            pltpu.SemaphoreType.DMA,
            pltpu.SemaphoreType.DMA((N_DEV - 1,)),
            pltpu.SemaphoreType.DMA((N_DEV - 1,)),
            pltpu.SemaphoreType.DMA((N_DEV - 1,)),
            pltpu.SemaphoreType.DMA((N_DEV - 1,)),
        ],
        compiler_params=pltpu.CompilerParams(
            collective_id=0, vmem_limit_bytes=100 * 1024 * 1024),
    )(x)


# device time: 748424 ns/iter; 1.0077x vs baseline; 1.0077x over previous
import jax
import jax.numpy as jnp
from jax import lax
from jax.experimental import pallas as pl
from jax.experimental.pallas import tpu as pltpu

N_DEV = 8


def kernel(x):
    m_per, n = x.shape
    half = m_per // 2

    def body(x_ref, out_ref, stage_ref, copy_sem,
             send_cw, recv_cw, send_ccw, recv_ccw):
        my = lax.axis_index("i")
        left = lax.rem(my + N_DEV - 1, N_DEV)
        right = lax.rem(my + 1, N_DEV)

        barrier = pltpu.get_barrier_semaphore()
        for nbr in (left, right):
            pl.semaphore_signal(barrier, inc=1, device_id=(nbr,),
                                device_id_type=pl.DeviceIdType.MESH)
        pl.semaphore_wait(barrier, 2)

        stage_ref[...] = x_ref[...].astype(out_ref.dtype)
        local = pltpu.make_async_copy(
            stage_ref, out_ref.at[pl.ds(my * m_per, m_per), :], copy_sem)
        local.start()

        sends = []
        for h in range(N_DEV - 1):
            o_cw = lax.rem(my - h + N_DEV, N_DEV)
            o_ccw = lax.rem(my + h, N_DEV)
            src_cw = (stage_ref.at[pl.ds(0, half), :] if h == 0
                      else out_ref.at[pl.ds(o_cw * m_per, half), :])
            src_ccw = (stage_ref.at[pl.ds(half, half), :] if h == 0
                       else out_ref.at[pl.ds(o_ccw * m_per + half, half), :])
            rdma_cw = pltpu.make_async_remote_copy(
                src_ref=src_cw,
                dst_ref=out_ref.at[pl.ds(o_cw * m_per, half), :],
                send_sem=send_cw.at[h], recv_sem=recv_cw.at[h],
                device_id=(right,), device_id_type=pl.DeviceIdType.MESH)
            rdma_ccw = pltpu.make_async_remote_copy(
                src_ref=src_ccw,
                dst_ref=out_ref.at[pl.ds(o_ccw * m_per + half, half), :],
                send_sem=send_ccw.at[h], recv_sem=recv_ccw.at[h],
                device_id=(left,), device_id_type=pl.DeviceIdType.MESH)
            rdma_cw.start()
            rdma_ccw.start()
            rdma_cw.wait_recv()
            rdma_ccw.wait_recv()
            sends.append(rdma_cw)
            sends.append(rdma_ccw)

        for rdma in sends:
            rdma.wait_send()
        local.wait()

    return pl.pallas_call(
        body,
        out_shape=jax.ShapeDtypeStruct((N_DEV * m_per, n), jnp.bfloat16),
        in_specs=[pl.BlockSpec(memory_space=pltpu.VMEM)],
        out_specs=pl.BlockSpec(memory_space=pl.ANY),
        scratch_shapes=[
            pltpu.VMEM((m_per, n), jnp.bfloat16),
            pltpu.SemaphoreType.DMA,
            pltpu.SemaphoreType.DMA((N_DEV - 1,)),
            pltpu.SemaphoreType.DMA((N_DEV - 1,)),
            pltpu.SemaphoreType.DMA((N_DEV - 1,)),
            pltpu.SemaphoreType.DMA((N_DEV - 1,)),
        ],
        compiler_params=pltpu.CompilerParams(
            collective_id=0, vmem_limit_bytes=100 * 1024 * 1024),
    )(x)


# device time: 734427 ns/iter; 1.0269x vs baseline; 1.0191x over previous
import jax
import jax.numpy as jnp
from jax import lax
from jax.experimental import pallas as pl
from jax.experimental.pallas import tpu as pltpu

N_DEV = 8
N_STREAM = 2


def kernel(x):
    m_per, n = x.shape
    half = m_per // 2
    quarter = half // N_STREAM

    def body(x_ref, out_ref, stage_ref, copy_sem, send_sems, recv_sems):
        my = lax.axis_index("i")
        left = lax.rem(my + N_DEV - 1, N_DEV)
        right = lax.rem(my + 1, N_DEV)

        barrier = pltpu.get_barrier_semaphore()
        for nbr in (left, right):
            pl.semaphore_signal(barrier, inc=1, device_id=(nbr,),
                                device_id_type=pl.DeviceIdType.MESH)
        pl.semaphore_wait(barrier, 2)

        def chain_rows(c, h):
            if c < N_STREAM:
                origin = lax.rem(my - h + N_DEV, N_DEV)
                off = c * quarter
            else:
                origin = lax.rem(my + h, N_DEV)
                off = half + (c - N_STREAM) * quarter
            return origin * m_per + off

        def make_rdma(c, h):
            row = chain_rows(c, h)
            src = (stage_ref.at[pl.ds(chain_rows(c, 0) - my * m_per, quarter), :]
                   if h == 0 else out_ref.at[pl.ds(row, quarter), :])
            tgt = right if c < N_STREAM else left
            return pltpu.make_async_remote_copy(
                src_ref=src,
                dst_ref=out_ref.at[pl.ds(row, quarter), :],
                send_sem=send_sems.at[c, h], recv_sem=recv_sems.at[c, h],
                device_id=(tgt,), device_id_type=pl.DeviceIdType.MESH)

        prev = [None] * (2 * N_STREAM)
        stage_ref[pl.ds(0, half), :] = x_ref[pl.ds(0, half), :].astype(
            out_ref.dtype)
        for c in range(N_STREAM):
            prev[c] = make_rdma(c, 0)
            prev[c].start()
        stage_ref[pl.ds(half, half), :] = x_ref[pl.ds(half, half), :].astype(
            out_ref.dtype)
        for c in range(N_STREAM, 2 * N_STREAM):
            prev[c] = make_rdma(c, 0)
            prev[c].start()
        local = pltpu.make_async_copy(
            stage_ref, out_ref.at[pl.ds(my * m_per, m_per), :], copy_sem)
        local.start()

        sends = list(prev)
        for h in range(1, N_DEV - 1):
            for c in range(2 * N_STREAM):
                prev[c].wait_recv()
                prev[c] = make_rdma(c, h)
                prev[c].start()
                sends.append(prev[c])
        for c in range(2 * N_STREAM):
            prev[c].wait_recv()
        for rdma in sends:
            rdma.wait_send()
        local.wait()

    return pl.pallas_call(
        body,
        out_shape=jax.ShapeDtypeStruct((N_DEV * m_per, n), jnp.bfloat16),
        in_specs=[pl.BlockSpec(memory_space=pltpu.VMEM)],
        out_specs=pl.BlockSpec(memory_space=pl.ANY),
        scratch_shapes=[
            pltpu.VMEM((m_per, n), jnp.bfloat16),
            pltpu.SemaphoreType.DMA,
            pltpu.SemaphoreType.DMA((2 * N_STREAM, N_DEV - 1)),
            pltpu.SemaphoreType.DMA((2 * N_STREAM, N_DEV - 1)),
        ],
        compiler_params=pltpu.CompilerParams(
            collective_id=0, vmem_limit_bytes=100 * 1024 * 1024),
    )(x)


# device time: 727455 ns/iter; 1.0368x vs baseline; 1.0096x over previous
import jax
import jax.numpy as jnp
from jax import lax
from jax.experimental import pallas as pl
from jax.experimental.pallas import tpu as pltpu

N_DEV = 8
N_STREAM = 2


def kernel(x):
    m_per, n = x.shape
    half = m_per // 2
    quarter = half // N_STREAM

    def body(x_ref, out_ref, xv_ref, stage_ref, load_sems, copy_sem,
             send_sems, recv_sems):
        my = lax.axis_index("i")
        left = lax.rem(my + N_DEV - 1, N_DEV)
        right = lax.rem(my + 1, N_DEV)

        barrier = pltpu.get_barrier_semaphore()
        for nbr in (left, right):
            pl.semaphore_signal(barrier, inc=1, device_id=(nbr,),
                                device_id_type=pl.DeviceIdType.MESH)
        pl.semaphore_wait(barrier, 2)

        def chain_rows(c, h):
            if c < N_STREAM:
                origin = lax.rem(my - h + N_DEV, N_DEV)
                off = c * quarter
            else:
                origin = lax.rem(my + h, N_DEV)
                off = half + (c - N_STREAM) * quarter
            return origin * m_per + off

        def make_rdma(c, h):
            row = chain_rows(c, h)
            src = (stage_ref.at[pl.ds(chain_rows(c, 0) - my * m_per, quarter), :]
                   if h == 0 else out_ref.at[pl.ds(row, quarter), :])
            tgt = right if c < N_STREAM else left
            return pltpu.make_async_remote_copy(
                src_ref=src,
                dst_ref=out_ref.at[pl.ds(row, quarter), :],
                send_sem=send_sems.at[c, h], recv_sem=recv_sems.at[c, h],
                device_id=(tgt,), device_id_type=pl.DeviceIdType.MESH)

        loads = []
        for c in range(2 * N_STREAM):
            ld = pltpu.make_async_copy(
                x_ref.at[pl.ds(c * quarter, quarter), :],
                xv_ref.at[pl.ds(c * quarter, quarter), :],
                load_sems.at[c])
            ld.start()
            loads.append(ld)
        prev = [None] * (2 * N_STREAM)
        for c in range(2 * N_STREAM):
            loads[c].wait()
            sl = pl.ds(c * quarter, quarter)
            stage_ref[sl, :] = xv_ref[sl, :].astype(out_ref.dtype)
            prev[c] = make_rdma(c, 0)
            prev[c].start()
        local = pltpu.make_async_copy(
            stage_ref, out_ref.at[pl.ds(my * m_per, m_per), :], copy_sem)
        local.start()

        sends = list(prev)
        for h in range(1, N_DEV - 1):
            for c in range(2 * N_STREAM):
                prev[c].wait_recv()
                prev[c] = make_rdma(c, h)
                prev[c].start()
                sends.append(prev[c])
        for c in range(2 * N_STREAM):
            prev[c].wait_recv()
        for rdma in sends:
            rdma.wait_send()
        local.wait()

    return pl.pallas_call(
        body,
        out_shape=jax.ShapeDtypeStruct((N_DEV * m_per, n), jnp.bfloat16),
        in_specs=[pl.BlockSpec(memory_space=pl.ANY)],
        out_specs=pl.BlockSpec(memory_space=pl.ANY),
        scratch_shapes=[
            pltpu.VMEM((m_per, n), jnp.float32),
            pltpu.VMEM((m_per, n), jnp.bfloat16),
            pltpu.SemaphoreType.DMA((2 * N_STREAM,)),
            pltpu.SemaphoreType.DMA,
            pltpu.SemaphoreType.DMA((2 * N_STREAM, N_DEV - 1)),
            pltpu.SemaphoreType.DMA((2 * N_STREAM, N_DEV - 1)),
        ],
        compiler_params=pltpu.CompilerParams(
            collective_id=0, vmem_limit_bytes=100 * 1024 * 1024),
    )(x)


# device time: 727130 ns/iter; 1.0372x vs baseline; 1.0004x over previous
import jax
import jax.numpy as jnp
from jax import lax
from jax.experimental import pallas as pl
from jax.experimental.pallas import tpu as pltpu

N_DEV = 8
N_STREAM = 2


def kernel(x):
    m_per, n = x.shape
    half = m_per // 2
    quarter = half // N_STREAM

    def body(x_ref, out_ref, xv_ref, stage_ref, load_sems, copy_sem,
             send_sems, recv_sems):
        my = lax.axis_index("i")
        left = lax.rem(my + N_DEV - 1, N_DEV)
        right = lax.rem(my + 1, N_DEV)

        barrier = pltpu.get_barrier_semaphore()
        for nbr in (left, right):
            pl.semaphore_signal(barrier, inc=1, device_id=(nbr,),
                                device_id_type=pl.DeviceIdType.MESH)
        pl.semaphore_wait(barrier, 2)

        def chain_rows(c, h):
            if c < N_STREAM:
                origin = lax.rem(my - h + N_DEV, N_DEV)
                off = c * quarter
            else:
                origin = lax.rem(my + h, N_DEV)
                off = half + (c - N_STREAM) * quarter
            return origin * m_per + off

        def make_rdma(c, h):
            row = chain_rows(c, h)
            src = (stage_ref.at[pl.ds(chain_rows(c, 0) - my * m_per, quarter), :]
                   if h == 0 else out_ref.at[pl.ds(row, quarter), :])
            tgt = right if c < N_STREAM else left
            return pltpu.make_async_remote_copy(
                src_ref=src,
                dst_ref=out_ref.at[pl.ds(row, quarter), :],
                send_sem=send_sems.at[c, h], recv_sem=recv_sems.at[c, h],
                device_id=(tgt,), device_id_type=pl.DeviceIdType.MESH)

        startup = [0, N_STREAM, 1, N_STREAM + 1]
        loads = {}
        for c in startup:
            ld = pltpu.make_async_copy(
                x_ref.at[pl.ds(c * quarter, quarter), :],
                xv_ref.at[pl.ds(c * quarter, quarter), :],
                load_sems.at[c])
            ld.start()
            loads[c] = ld
        prev = [None] * (2 * N_STREAM)
        for c in startup:
            loads[c].wait()
            sl = pl.ds(c * quarter, quarter)
            stage_ref[sl, :] = xv_ref[sl, :].astype(out_ref.dtype)
            prev[c] = make_rdma(c, 0)
            prev[c].start()
        local = pltpu.make_async_copy(
            stage_ref, out_ref.at[pl.ds(my * m_per, m_per), :], copy_sem)
        local.start()

        sends = list(prev)
        for h in range(1, N_DEV - 1):
            for c in range(2 * N_STREAM):
                prev[c].wait_recv()
                prev[c] = make_rdma(c, h)
                prev[c].start()
                sends.append(prev[c])
        for c in range(2 * N_STREAM):
            prev[c].wait_recv()
        for rdma in sends:
            rdma.wait_send()
        local.wait()

    return pl.pallas_call(
        body,
        out_shape=jax.ShapeDtypeStruct((N_DEV * m_per, n), jnp.bfloat16),
        in_specs=[pl.BlockSpec(memory_space=pl.ANY)],
        out_specs=pl.BlockSpec(memory_space=pl.ANY),
        scratch_shapes=[
            pltpu.VMEM((m_per, n), jnp.float32),
            pltpu.VMEM((m_per, n), jnp.bfloat16),
            pltpu.SemaphoreType.DMA((2 * N_STREAM,)),
            pltpu.SemaphoreType.DMA,
            pltpu.SemaphoreType.DMA((2 * N_STREAM, N_DEV - 1)),
            pltpu.SemaphoreType.DMA((2 * N_STREAM, N_DEV - 1)),
        ],
        compiler_params=pltpu.CompilerParams(
            collective_id=0, vmem_limit_bytes=100 * 1024 * 1024),
    )(x)


# device time: 527795 ns/iter; 1.4290x vs baseline; 1.3777x over previous
import jax
import jax.numpy as jnp
from jax import lax
from jax.experimental import pallas as pl
from jax.experimental.pallas import tpu as pltpu

N_DEV = 8
NC = 3


def _coords(l):
    p = lax.rem(l, 4)
    g = p ^ (p >> 1)
    return g & 1, g >> 1, l // 4


def _logical(x, y, z):
    g = x | (y << 1)
    return z * 4 + (g ^ (g >> 1))


def kernel(x):
    m_per, n = x.shape
    base = (m_per // NC) // 32 * 32
    sizes = [base, base, m_per - 2 * base]
    offs = [0, base, 2 * base]

    def body(x_ref, out_ref, xv_ref, stage_ref, load_sems, copy_sem,
             send_sems, recv_sems):
        my = lax.axis_index("i")
        mx, my_y, mz = _coords(my)

        def partner(d):
            return _logical(mx ^ int(d == 0), my_y ^ int(d == 1),
                            mz ^ int(d == 2))

        def origin(fx, fy, fz):
            return _logical(mx ^ fx, my_y ^ fy, mz ^ fz)

        barrier = pltpu.get_barrier_semaphore()
        for d in range(NC):
            pl.semaphore_signal(barrier, inc=1, device_id=(partner(d),),
                                device_id_type=pl.DeviceIdType.MESH)
        pl.semaphore_wait(barrier, NC)

        def make_rdma(c, k, j, flips):
            d = (c + k) % NC
            o = origin(*flips)
            row = o * m_per + offs[c]
            if flips == (0, 0, 0):
                src = stage_ref.at[pl.ds(offs[c], sizes[c]), :]
            else:
                src = out_ref.at[pl.ds(row, sizes[c]), :]
            idx = (1 << k) - 1 + j
            return pltpu.make_async_remote_copy(
                src_ref=src,
                dst_ref=out_ref.at[pl.ds(row, sizes[c]), :],
                send_sem=send_sems.at[c, idx], recv_sem=recv_sems.at[c, idx],
                device_id=(partner(d),), device_id_type=pl.DeviceIdType.MESH)

        def stage_flips(c, k):
            combos = [(0, 0, 0)]
            for kk in range(k):
                d = (c + kk) % NC
                combos = combos + [
                    tuple(f ^ int(dd == d) for dd, f in enumerate(fl))
                    for fl in combos]
            return combos

        loads = []
        for c in range(NC):
            ld = pltpu.make_async_copy(
                x_ref.at[pl.ds(offs[c], sizes[c]), :],
                xv_ref.at[pl.ds(offs[c], sizes[c]), :],
                load_sems.at[c])
            ld.start()
            loads.append(ld)
        prev = [[] for _ in range(NC)]
        sends = []
        for c in range(NC):
            loads[c].wait()
            sl = pl.ds(offs[c], sizes[c])
            stage_ref[sl, :] = xv_ref[sl, :].astype(out_ref.dtype)
            r = make_rdma(c, 0, 0, (0, 0, 0))
            r.start()
            prev[c] = [r]
            sends.append(r)
        local = pltpu.make_async_copy(
            stage_ref, out_ref.at[pl.ds(my * m_per, m_per), :], copy_sem)
        local.start()

        for k in range(1, NC):
            for c in range(NC):
                for r in prev[c]:
                    r.wait_recv()
                prev[c] = []
                for j, flips in enumerate(stage_flips(c, k)):
                    r = make_rdma(c, k, j, flips)
                    r.start()
                    prev[c].append(r)
                    sends.append(r)
        for c in range(NC):
            for r in prev[c]:
                r.wait_recv()
        for r in sends:
            r.wait_send()
        local.wait()

    n_sems = (1 << NC) - 1
    return pl.pallas_call(
        body,
        out_shape=jax.ShapeDtypeStruct((N_DEV * m_per, n), jnp.bfloat16),
        in_specs=[pl.BlockSpec(memory_space=pl.ANY)],
        out_specs=pl.BlockSpec(memory_space=pl.ANY),
        scratch_shapes=[
            pltpu.VMEM((m_per, n), jnp.float32),
            pltpu.VMEM((m_per, n), jnp.bfloat16),
            pltpu.SemaphoreType.DMA((NC,)),
            pltpu.SemaphoreType.DMA,
            pltpu.SemaphoreType.DMA((NC, n_sems)),
            pltpu.SemaphoreType.DMA((NC, n_sems)),
        ],
        compiler_params=pltpu.CompilerParams(
            collective_id=0, vmem_limit_bytes=100 * 1024 * 1024),
    )(x)


# device time: 519130 ns/iter; 1.4528x vs baseline; 1.0167x over previous
import jax
import jax.numpy as jnp
from jax import lax
from jax.experimental import pallas as pl
from jax.experimental.pallas import tpu as pltpu

N_DEV = 8
NC = 3


def _coords(l):
    p = lax.rem(l, 4)
    g = p ^ (p >> 1)
    return g & 1, g >> 1, l // 4


def _logical(x, y, z):
    g = x | (y << 1)
    return z * 4 + (g ^ (g >> 1))


def kernel(x):
    m_per, n = x.shape
    base = (m_per // NC) // 32 * 32
    sizes = [base, base, m_per - 2 * base]
    offs = [0, base, 2 * base]

    def body(x_ref, out_ref, xv_ref, stage_ref, load_sems, copy_sem,
             send_sems, recv_sems):
        my = lax.axis_index("i")
        mx, my_y, mz = _coords(my)

        def partner(d):
            return _logical(mx ^ int(d == 0), my_y ^ int(d == 1),
                            mz ^ int(d == 2))

        def origin(fx, fy, fz):
            return _logical(mx ^ fx, my_y ^ fy, mz ^ fz)

        barrier = pltpu.get_barrier_semaphore()
        for d in range(NC):
            pl.semaphore_signal(barrier, inc=1, device_id=(partner(d),),
                                device_id_type=pl.DeviceIdType.MESH)
        pl.semaphore_wait(barrier, NC)

        def make_rdma(c, k, j, flips):
            d = (c + k) % NC
            o = origin(*flips)
            row = o * m_per + offs[c]
            if flips == (0, 0, 0):
                src = stage_ref.at[pl.ds(offs[c], sizes[c]), :]
            else:
                src = out_ref.at[pl.ds(row, sizes[c]), :]
            idx = (1 << k) - 1 + j
            return pltpu.make_async_remote_copy(
                src_ref=src,
                dst_ref=out_ref.at[pl.ds(row, sizes[c]), :],
                send_sem=send_sems.at[c, idx], recv_sem=recv_sems.at[c, idx],
                device_id=(partner(d),), device_id_type=pl.DeviceIdType.MESH)

        def stage_flips(c, k):
            combos = [(0, 0, 0)]
            for kk in range(k):
                d = (c + kk) % NC
                combos = combos + [
                    tuple(f ^ int(dd == d) for dd, f in enumerate(fl))
                    for fl in combos]
            return combos

        loads = []
        for c in range(NC):
            ld = pltpu.make_async_copy(
                x_ref.at[pl.ds(offs[c], sizes[c]), :],
                xv_ref.at[pl.ds(offs[c], sizes[c]), :],
                load_sems.at[c])
            ld.start()
            loads.append(ld)
        s0 = [None] * NC
        s1 = [[] for _ in range(NC)]
        s2 = [[] for _ in range(NC)]
        sends = []

        def issue(c, k, j, flips, bucket):
            r = make_rdma(c, k, j, flips)
            r.start()
            bucket.append(r)
            sends.append(r)

        for c in range(NC):
            loads[c].wait()
            sl = pl.ds(offs[c], sizes[c])
            stage_ref[sl, :] = xv_ref[sl, :].astype(out_ref.dtype)
            own = (0, 0, 0)
            r = make_rdma(c, 0, 0, own)
            r.start()
            s0[c] = r
            sends.append(r)
            issue(c, 1, 0, own, s1[c])
            issue(c, 2, 0, own, s2[c])
        local = pltpu.make_async_copy(
            stage_ref, out_ref.at[pl.ds(my * m_per, m_per), :], copy_sem)
        local.start()

        for c in range(NC):
            f0 = stage_flips(c, 1)[1]
            s0[c].wait_recv()
            issue(c, 1, 1, f0, s1[c])
            issue(c, 2, 1, f0, s2[c])

        for c in range(NC):
            flips = stage_flips(c, 2)
            for r in s1[c]:
                r.wait_recv()
            issue(c, 2, 2, flips[2], s2[c])
            issue(c, 2, 3, flips[3], s2[c])

        for c in range(NC):
            for r in s2[c]:
                r.wait_recv()
        for r in sends:
            r.wait_send()
        local.wait()

    n_sems = (1 << NC) - 1
    return pl.pallas_call(
        body,
        out_shape=jax.ShapeDtypeStruct((N_DEV * m_per, n), jnp.bfloat16),
        in_specs=[pl.BlockSpec(memory_space=pl.ANY)],
        out_specs=pl.BlockSpec(memory_space=pl.ANY),
        scratch_shapes=[
            pltpu.VMEM((m_per, n), jnp.float32),
            pltpu.VMEM((m_per, n), jnp.bfloat16),
            pltpu.SemaphoreType.DMA((NC,)),
            pltpu.SemaphoreType.DMA,
            pltpu.SemaphoreType.DMA((NC, n_sems)),
            pltpu.SemaphoreType.DMA((NC, n_sems)),
        ],
        compiler_params=pltpu.CompilerParams(
            collective_id=0, vmem_limit_bytes=100 * 1024 * 1024),
    )(x)
